# baseline (device time: 18631 ns/iter reference)
import jax
import jax.numpy as jnp
from jax import lax
from jax.experimental import pallas as pl
from jax.experimental.pallas import tpu as pltpu

N_DEV = 8
M = 256
NCOLS = 128
GCOLS = NCOLS // N_DEV
T = M // 8


def _roll(x, shift, axis):
    return pltpu.roll(x, shift, axis)


def _bitswap(y):
    row = lax.broadcasted_iota(jnp.int32, (M, 1), 0)
    lane = lax.broadcasted_iota(jnp.int32, (1, NCOLS), 1)
    for b in range(3):
        sb = 1 << b
        lb = GCOLS << b
        rb = (row >> b) & 1
        cb = (lane >> (4 + b)) & 1
        same = rb == cb
        r1 = _roll(_roll(y, NCOLS - lb, 1), sb, 0)
        r2 = _roll(_roll(y, lb, 1), M - sb, 0)
        y = jnp.where(same, y, jnp.where(rb == 1, r1, r2))
    return y


def _zstage(y, k, j, rank_map):
    up_full = (rank_map & k) == 0
    if j >= 8:
        jr = j // 256 if j >= 256 else j
        g = M // (2 * jr)
        yr = y.reshape(g, 2 * jr, NCOLS)
        lo, hi = yr[:, :jr, :], yr[:, jr:, :]
        mn, mx = jnp.minimum(lo, hi), jnp.maximum(lo, hi)
        up = jnp.broadcast_to(up_full, (M, NCOLS)).reshape(g, 2 * jr, NCOLS)[:, :jr, :]
        return jnp.concatenate(
            [jnp.where(up, mn, mx), jnp.where(up, mx, mn)], axis=1
        ).reshape(M, NCOLS)
    shift = GCOLS * j
    lane = lax.broadcasted_iota(jnp.int32, (1, NCOLS), 1)
    is_low = ((lane >> 4) & j) == 0
    partner = jnp.where(
        is_low, _roll(y, NCOLS - shift, 1), _roll(y, shift, 1)
    )
    keep_min = up_full == is_low
    return jnp.where(keep_min, jnp.minimum(y, partner), jnp.maximum(y, partner))


def _rank_maps(row0):
    row = lax.broadcasted_iota(jnp.int32, (M, 1), 0)
    lane = lax.broadcasted_iota(jnp.int32, (1, NCOLS), 1)
    tu = (row >> 3) * 8 + (lane >> 4)
    return row0 + tu, (row % 8) * M + tu


def _sort_local(y, rank1):
    k = 2
    while k <= M:
        j = k // 2
        while j >= 1:
            y = _zstage(y, k, j, rank1)
            j //= 2
        k *= 2
    return y


def _merge(y, rank2):
    for k in (2 * M, 4 * M, 8 * M):
        j = k // 2
        while j >= 1:
            y = _zstage(y, k, j, rank2)
            j //= 2
    return y


def kernel(x):
    assert x.shape == (M, NCOLS)

    def body(
        x_ref, out_ref,
        stage1, recv1, stage2, recv2,
        send_sems1, recv_sems1, send_sems2, recv_sems2, local_sems,
    ):
        my = lax.axis_index("i")
        rank1, rank2 = _rank_maps(my * M)

        barrier_sem = pltpu.get_barrier_semaphore()
        for off in range(1, N_DEV):
            pl.semaphore_signal(
                barrier_sem, inc=1,
                device_id=(my ^ off,), device_id_type=pl.DeviceIdType.MESH,
            )
        pl.semaphore_wait(barrier_sem, N_DEV - 1)

        z = _sort_local(_bitswap(x_ref[...]), rank1)
        stage1[...] = z.reshape(T, N_DEV, NCOLS)

        self1 = pltpu.make_async_copy(
            stage1.at[:, my], recv1.at[:, my], local_sems.at[0]
        )
        self1.start()
        rdmas = []
        for off in range(1, N_DEV):
            tgt = my ^ off
            rdma = pltpu.make_async_remote_copy(
                src_ref=stage1.at[:, tgt],
                dst_ref=recv1.at[:, my],
                send_sem=send_sems1.at[off - 1],
                recv_sem=recv_sems1.at[off - 1],
                device_id=(tgt,),
                device_id_type=pl.DeviceIdType.MESH,
            )
            rdma.start()
            rdmas.append(rdma)
        self1.wait()
        for rdma in rdmas[-(N_DEV - 1):]:
            rdma.wait_recv()

        w = _merge(recv1[...].reshape(M, NCOLS), rank2)
        stage2[...] = w.reshape(T, N_DEV, NCOLS)

        self2 = pltpu.make_async_copy(
            stage2.at[:, my], recv2.at[:, my], local_sems.at[1]
        )
        self2.start()
        for off in range(1, N_DEV):
            tgt = my ^ off
            rdma = pltpu.make_async_remote_copy(
                src_ref=stage2.at[:, tgt],
                dst_ref=recv2.at[:, my],
                send_sem=send_sems2.at[off - 1],
                recv_sem=recv_sems2.at[off - 1],
                device_id=(tgt,),
                device_id_type=pl.DeviceIdType.MESH,
            )
            rdma.start()
            rdmas.append(rdma)
        self2.wait()
        for rdma in rdmas[-(N_DEV - 1):]:
            rdma.wait_recv()

        out_ref[...] = _bitswap(recv2[...].reshape(M, NCOLS))
        for rdma in rdmas:
            rdma.wait_send()

    return pl.pallas_call(
        body,
        out_shape=jax.ShapeDtypeStruct((M, NCOLS), x.dtype),
        in_specs=[pl.BlockSpec(memory_space=pltpu.VMEM)],
        out_specs=pl.BlockSpec(memory_space=pltpu.VMEM),
        scratch_shapes=[
            pltpu.VMEM((T, N_DEV, NCOLS), x.dtype),
            pltpu.VMEM((T, N_DEV, NCOLS), x.dtype),
            pltpu.VMEM((T, N_DEV, NCOLS), x.dtype),
            pltpu.VMEM((T, N_DEV, NCOLS), x.dtype),
            pltpu.SemaphoreType.DMA((N_DEV - 1,)),
            pltpu.SemaphoreType.DMA((N_DEV - 1,)),
            pltpu.SemaphoreType.DMA((N_DEV - 1,)),
            pltpu.SemaphoreType.DMA((N_DEV - 1,)),
            pltpu.SemaphoreType.DMA((2,)),
        ],
        compiler_params=pltpu.CompilerParams(collective_id=0),
    )(x)
